# baseline (speedup 1.0000x reference)
import functools

import jax
import jax.numpy as jnp
import numpy as np
from jax import lax
from jax.experimental import pallas as pl
from jax.experimental.pallas import tpu as pltpu
from jax.experimental.pallas import tpu_sc as plsc

_ATOM_DIMS = [119, 5, 12, 12, 10, 6, 6, 2, 2]
_TOTAL = int(np.sum(_ATOM_DIMS))
_N = 10000
_DEG = 16
_D = 128
_L = 4

_NC = 2
_NS = 16
_NW = _NC * _NS
_NPW = 320
_NPAD = _NW * _NPW
_CH = 4
_NCH = _NPW // _CH
_EPC = _CH * _DEG
_EPAD = _NPAD * _DEG
_EROWS = _EPAD // _EPC
_NB = 3
_ASH = 10112


def _embed_pre_body(hidx_ref, table_ref, w_ref, w1_ref, w2_ref, b_ref,
                    h0_ref, ew_ref, sw_ref, a_ref, bb_ref):
    hidx = hidx_ref[...]
    lanes = lax.broadcasted_iota(jnp.int32, (1, 256), 1)
    oh = jnp.zeros((_N, 256), jnp.float32)
    off = 0
    for j, dj in enumerate(_ATOM_DIMS):
        col = hidx[:, j:j + 1] + off
        oh = oh + (col == lanes).astype(jnp.float32)
        off += dj
    h0 = jnp.dot(oh, table_ref[...], preferred_element_type=jnp.float32)
    h0_ref[...] = h0
    w = w_ref[...]
    wabs = jnp.sum(jnp.abs(w), axis=1, keepdims=True)
    ew = w / (wabs + 1e-8)
    ew_ref[...] = ew
    sw_ref[...] = jnp.sum(ew, axis=1, keepdims=True)
    a_ref[0:_N, :] = jnp.dot(h0, w1_ref[...], preferred_element_type=jnp.float32)
    bb_ref[...] = jnp.dot(h0, w2_ref[...], preferred_element_type=jnp.float32) + b_ref[...]


_embed_pre = pl.pallas_call(
    _embed_pre_body,
    out_shape=(
        jax.ShapeDtypeStruct((_N, _D), jnp.float32),
        jax.ShapeDtypeStruct((_N, _DEG), jnp.float32),
        jax.ShapeDtypeStruct((_N, 1), jnp.float32),
        jax.ShapeDtypeStruct((_ASH, _D), jnp.float32),
        jax.ShapeDtypeStruct((_N, _D), jnp.float32),
    ),
)


def _agg_body(a_hbm, srcp_hbm, ewp_hbm, s1_hbm, s2_hbm, s3_hbm,
              a_sh, idx_v, ewv,
              rows0, rows1, rows2,
              o10, o20, o30, o11, o21, o31, o12, o22, o32,
              semg0, semg1, semg2, semo0, semo1, semo2):
    cid = lax.axis_index("c")
    sid = lax.axis_index("s")
    wid = sid * _NC + cid
    rb = wid * _NCH
    stg = _ASH // _NS
    pltpu.sync_copy(a_hbm.at[pl.ds(sid * stg, stg)], a_sh.at[pl.ds(sid * stg, stg)])
    pltpu.sync_copy(srcp_hbm.at[pl.ds(rb, _NCH)], idx_v)
    pltpu.sync_copy(ewp_hbm.at[pl.ds(rb, _NCH)], ewv)
    plsc.subcore_barrier()

    def start_gather(j, rows, semg):
        pltpu.async_copy(a_sh.at[idx_v.at[j]], rows, semg)

    def wait_gather(rows, semg):
        pltpu.make_async_copy(a_sh.at[idx_v.at[0]], rows, semg).wait()

    def compute(j, rows, o1, o2, o3):
        def node_body(i, c2):
            e0 = i * _DEG
            wvec = ewv[j, pl.ds(e0, _DEG)]
            ws = [wvec[k] for k in range(_DEG)]
            for cc in range(_D // 16):
                sl = pl.ds(cc * 16, 16)
                r0 = rows[e0, sl]
                acc_s = r0
                acc_m = r0
                acc_w = r0 * ws[0]
                for k in range(1, _DEG):
                    r = rows[e0 + k, sl]
                    acc_s = acc_s + r
                    acc_m = jnp.maximum(acc_m, r)
                    acc_w = acc_w + r * ws[k]
                o1[i, sl] = acc_s
                o2[i, sl] = acc_m
                o3[i, sl] = acc_w
            return c2

        lax.fori_loop(0, _CH, node_body, 0, unroll=2)

    def store_out(j, o1, o2, o3, semo):
        nb = wid * _NPW + j * _CH
        pltpu.async_copy(o1, s1_hbm.at[pl.ds(nb, _CH)], semo)
        pltpu.async_copy(o2, s2_hbm.at[pl.ds(nb, _CH)], semo)
        pltpu.async_copy(o3, s3_hbm.at[pl.ds(nb, _CH)], semo)

    def wait_out(o1, o2, o3, semo):
        pltpu.make_async_copy(o1, s1_hbm.at[pl.ds(0, _CH)], semo).wait()
        pltpu.make_async_copy(o2, s2_hbm.at[pl.ds(0, _CH)], semo).wait()
        pltpu.make_async_copy(o3, s3_hbm.at[pl.ds(0, _CH)], semo).wait()

    bufs = ((rows0, o10, o20, o30, semg0, semo0),
            (rows1, o11, o21, o31, semg1, semo1),
            (rows2, o12, o22, o32, semg2, semo2))

    for p in range(_NB):
        start_gather(p, bufs[p][0], bufs[p][4])

    def ring_body(jj, carry):
        for p in range(_NB):
            rows, o1, o2, o3, semg, semo = bufs[p]
            j = _NB * jj + p
            wait_gather(rows, semg)

            @pl.when(jj > 0)
            def _():
                wait_out(o1, o2, o3, semo)

            compute(j, rows, o1, o2, o3)
            store_out(j, o1, o2, o3, semo)

            @pl.when(j + _NB < _NCH)
            def _():
                start_gather(j + _NB, rows, semg)

        return carry

    lax.fori_loop(0, _NCH // _NB, ring_body, 0)
    for j in range((_NCH // _NB) * _NB, _NCH):
        p = j % _NB
        rows, o1, o2, o3, semg, semo = bufs[p]
        wait_gather(rows, semg)
        wait_out(o1, o2, o3, semo)
        compute(j, rows, o1, o2, o3)
        store_out(j, o1, o2, o3, semo)
    for p in range(_NB):
        rows, o1, o2, o3, semg, semo = bufs[p]
        wait_out(o1, o2, o3, semo)


_agg = functools.partial(
    pl.kernel,
    mesh=plsc.VectorSubcoreMesh(core_axis_name="c", subcore_axis_name="s"),
    out_type=[jax.ShapeDtypeStruct((_NPAD, _D), jnp.float32)] * 3,
    scratch_types=[
        pltpu.VMEM_SHARED((_ASH, _D), jnp.float32),
        pltpu.VMEM((_NCH, _EPC), jnp.int32),
        pltpu.VMEM((_NCH, _EPC), jnp.float32),
        pltpu.VMEM((_EPC, _D), jnp.float32),
        pltpu.VMEM((_EPC, _D), jnp.float32),
        pltpu.VMEM((_EPC, _D), jnp.float32),
        pltpu.VMEM((_CH, _D), jnp.float32),
        pltpu.VMEM((_CH, _D), jnp.float32),
        pltpu.VMEM((_CH, _D), jnp.float32),
        pltpu.VMEM((_CH, _D), jnp.float32),
        pltpu.VMEM((_CH, _D), jnp.float32),
        pltpu.VMEM((_CH, _D), jnp.float32),
        pltpu.VMEM((_CH, _D), jnp.float32),
        pltpu.VMEM((_CH, _D), jnp.float32),
        pltpu.VMEM((_CH, _D), jnp.float32),
        pltpu.SemaphoreType.DMA,
        pltpu.SemaphoreType.DMA,
        pltpu.SemaphoreType.DMA,
        pltpu.SemaphoreType.DMA,
        pltpu.SemaphoreType.DMA,
        pltpu.SemaphoreType.DMA,
    ],
)(_agg_body)


def _post_core(h_ref, bb_ref, s1_ref, s2_ref, s3_ref, sw_ref, snorm_ref,
               pw_ref, pb_ref, g_ref, beta_ref):
    h = h_ref[...]
    bb = bb_ref[...]
    s1 = s1_ref[0:_N, :]
    s2 = s2_ref[0:_N, :]
    s3 = s3_ref[0:_N, :]
    sw = sw_ref[...]
    pw = pw_ref[...]
    sum_e = s1 + 16.0 * bb
    max_agg = s2 + bb
    dir_agg = jnp.abs(s3 + sw * bb - sw * h)
    p_ms = pw[128:256, :] * (1.0 / 16.0) + pw[256:384, :]
    x = (jnp.dot(h, pw[0:128, :], preferred_element_type=jnp.float32)
         + jnp.dot(sum_e, p_ms, preferred_element_type=jnp.float32)
         + jnp.dot(max_agg, pw[384:512, :], preferred_element_type=jnp.float32)
         + jnp.dot(dir_agg, pw[512:640, :], preferred_element_type=jnp.float32)
         + pb_ref[...])
    x = x * snorm_ref[...]
    mu = jnp.mean(x, axis=0, keepdims=True)
    xc = x - mu
    var = jnp.mean(xc * xc, axis=0, keepdims=True)
    y = xc * lax.rsqrt(var + 1e-5) * g_ref[...] + beta_ref[...]
    return jnp.maximum(y, 0.0)


def _post_body(h_ref, bb_ref, s1_ref, s2_ref, s3_ref, sw_ref, snorm_ref,
               pw_ref, pb_ref, g_ref, beta_ref, out_ref):
    out_ref[...] = _post_core(h_ref, bb_ref, s1_ref, s2_ref, s3_ref, sw_ref,
                              snorm_ref, pw_ref, pb_ref, g_ref, beta_ref)


_post = pl.pallas_call(
    _post_body,
    out_shape=jax.ShapeDtypeStruct((_N, _D), jnp.float32),
)


def _pre_body(h_ref, w1_ref, w2_ref, b_ref, a_ref, bb_ref):
    h = h_ref[...]
    a_ref[0:_N, :] = jnp.dot(h, w1_ref[...], preferred_element_type=jnp.float32)
    bb_ref[...] = jnp.dot(h, w2_ref[...], preferred_element_type=jnp.float32) + b_ref[...]


_pre = pl.pallas_call(
    _pre_body,
    out_shape=(
        jax.ShapeDtypeStruct((_ASH, _D), jnp.float32),
        jax.ShapeDtypeStruct((_N, _D), jnp.float32),
    ),
)


def _post_ro_body(h_ref, bb_ref, s1_ref, s2_ref, s3_ref, sw_ref, snorm_ref,
                  pw_ref, pb_ref, g_ref, beta_ref,
                  w0_ref, b0_ref, w1_ref, b1_ref, w2_ref, b2_ref, out_ref):
    hn = _post_core(h_ref, bb_ref, s1_ref, s2_ref, s3_ref, sw_ref, snorm_ref,
                    pw_ref, pb_ref, g_ref, beta_ref)
    hg = jnp.mean(hn, axis=0, keepdims=True)
    y = jnp.maximum(jnp.dot(hg, w0_ref[...], preferred_element_type=jnp.float32) + b0_ref[...], 0.0)
    y = jnp.maximum(jnp.dot(y, w1_ref[...], preferred_element_type=jnp.float32) + b1_ref[...], 0.0)
    out_ref[...] = jnp.dot(y, w2_ref[...], preferred_element_type=jnp.float32) + b2_ref[...]


_post_ro = pl.pallas_call(
    _post_ro_body,
    out_shape=jax.ShapeDtypeStruct((1, 128), jnp.float32),
)


def kernel(h, edge_index, eig, snorm_n, atom_table, pre_W, pre_b, post_W,
           post_b, bn_g, bn_b, ro_W0, ro_b0, ro_W1, ro_b1, ro_W2, ro_b2):
    hidx = h.astype(jnp.int32)
    src = edge_index[0].astype(jnp.int32)
    srcp = jnp.pad(src, (0, _EPAD - src.shape[0])).reshape(_EROWS, _EPC)
    w_col = eig[:, 1].reshape(_N, _DEG)
    table_pad = jnp.pad(atom_table, ((0, 256 - _TOTAL), (0, 0)))

    def wslice(l):
        return (pre_W[l, :_D, :], pre_W[l, _D:, :], pre_b[l].reshape(1, _D))

    w1, w2, b = wslice(0)
    hcur, ew, sw, a, bb = _embed_pre(hidx, table_pad, w_col, w1, w2, b)
    ewp = jnp.pad(ew.reshape(-1), (0, _EPAD - _N * _DEG)).reshape(_EROWS, _EPC)

    for l in range(_L - 1):
        s1, s2, s3 = _agg(a, srcp, ewp)
        hcur = _post(hcur, bb, s1, s2, s3, sw, snorm_n, post_W[l],
                     post_b[l].reshape(1, _D), bn_g[l].reshape(1, _D),
                     bn_b[l].reshape(1, _D))
        w1, w2, b = wslice(l + 1)
        a, bb = _pre(hcur, w1, w2, b)

    s1, s2, s3 = _agg(a, srcp, ewp)
    l = _L - 1
    return _post_ro(hcur, bb, s1, s2, s3, sw, snorm_n, post_W[l],
                    post_b[l].reshape(1, _D), bn_g[l].reshape(1, _D),
                    bn_b[l].reshape(1, _D),
                    ro_W0, ro_b0.reshape(1, -1), ro_W1,
                    ro_b1.reshape(1, -1), ro_W2, ro_b2.reshape(1, -1))


# speedup vs baseline: 1.3303x; 1.3303x over previous
import functools

import jax
import jax.numpy as jnp
import numpy as np
from jax import lax
from jax.experimental import pallas as pl
from jax.experimental.pallas import tpu as pltpu
from jax.experimental.pallas import tpu_sc as plsc

_ATOM_DIMS = [119, 5, 12, 12, 10, 6, 6, 2, 2]
_TOTAL = int(np.sum(_ATOM_DIMS))
_N = 10000
_DEG = 16
_D = 128
_L = 4

_NC = 2
_NS = 16
_NW = _NC * _NS
_NPW = 320
_NPAD = _NW * _NPW
_CH = 4
_NCH = _NPW // _CH
_EPC = _CH * _DEG
_EPAD = _NPAD * _DEG
_EROWS = _EPAD // _EPC
_NB = 2
_ASH = 10112


def _embed_pre_body(hidx_ref, table_ref, w_ref, w1_ref, w2_ref, b_ref,
                    h0_ref, ew_ref, sw_ref, a_ref, bb_ref):
    hidx = hidx_ref[...]
    lanes = lax.broadcasted_iota(jnp.int32, (1, 256), 1)
    oh = jnp.zeros((_N, 256), jnp.float32)
    off = 0
    for j, dj in enumerate(_ATOM_DIMS):
        col = hidx[:, j:j + 1] + off
        oh = oh + (col == lanes).astype(jnp.float32)
        off += dj
    h0 = jnp.dot(oh, table_ref[...], preferred_element_type=jnp.float32)
    h0_ref[...] = h0
    w = w_ref[...]
    wabs = jnp.sum(jnp.abs(w), axis=1, keepdims=True)
    ew = w / (wabs + 1e-8)
    ew_ref[...] = ew
    sw_ref[...] = jnp.sum(ew, axis=1, keepdims=True)
    a_ref[0:_N, :] = jnp.dot(h0, w1_ref[...], preferred_element_type=jnp.float32)
    bb_ref[...] = jnp.dot(h0, w2_ref[...], preferred_element_type=jnp.float32) + b_ref[...]


_embed_pre = pl.pallas_call(
    _embed_pre_body,
    out_shape=(
        jax.ShapeDtypeStruct((_N, _D), jnp.float32),
        jax.ShapeDtypeStruct((_N, _DEG), jnp.float32),
        jax.ShapeDtypeStruct((_N, 1), jnp.float32),
        jax.ShapeDtypeStruct((_ASH, _D), jnp.float32),
        jax.ShapeDtypeStruct((_N, _D), jnp.float32),
    ),
)


def _agg_body(a_hbm, srcp_hbm, ewp_hbm, s1_hbm, s2_hbm, s3_hbm,
              a_sh, idx_v, ewv,
              rows0, rows1,
              o10, o20, o30, o11, o21, o31,
              semg0, semg1, semo0, semo1):
    cid = lax.axis_index("c")
    sid = lax.axis_index("s")
    wid = sid * _NC + cid
    rb = wid * _NCH
    stg = _ASH // _NS
    pltpu.sync_copy(a_hbm.at[pl.ds(sid * stg, stg)], a_sh.at[pl.ds(sid * stg, stg)])
    pltpu.sync_copy(srcp_hbm.at[pl.ds(rb, _NCH)], idx_v)
    pltpu.sync_copy(ewp_hbm.at[pl.ds(rb, _NCH)], ewv)
    plsc.subcore_barrier()

    def start_gather(j, rows, semg):
        pltpu.async_copy(a_sh.at[idx_v.at[j]], rows, semg)

    def wait_gather(rows, semg):
        pltpu.make_async_copy(a_sh.at[idx_v.at[0]], rows, semg).wait()

    def compute(j, rows, o1, o2, o3):
        def node_body(i, c2):
            e0 = i * _DEG
            wvec = ewv[j, pl.ds(e0, _DEG)]
            ws = [wvec[k] for k in range(_DEG)]
            for cc in range(_D // 16):
                sl = pl.ds(cc * 16, 16)
                r0 = rows[e0, sl]
                acc_s = r0
                acc_m = r0
                acc_w = r0 * ws[0]
                for k in range(1, _DEG):
                    r = rows[e0 + k, sl]
                    acc_s = acc_s + r
                    acc_m = jnp.maximum(acc_m, r)
                    acc_w = acc_w + r * ws[k]
                o1[i, sl] = acc_s
                o2[i, sl] = acc_m
                o3[i, sl] = acc_w
            return c2

        lax.fori_loop(0, _CH, node_body, 0)

    def store_out(j, o1, o2, o3, semo):
        nb = wid * _NPW + j * _CH
        pltpu.async_copy(o1, s1_hbm.at[pl.ds(nb, _CH)], semo)
        pltpu.async_copy(o2, s2_hbm.at[pl.ds(nb, _CH)], semo)
        pltpu.async_copy(o3, s3_hbm.at[pl.ds(nb, _CH)], semo)

    def wait_out(o1, o2, o3, semo):
        pltpu.make_async_copy(o1, s1_hbm.at[pl.ds(0, _CH)], semo).wait()
        pltpu.make_async_copy(o2, s2_hbm.at[pl.ds(0, _CH)], semo).wait()
        pltpu.make_async_copy(o3, s3_hbm.at[pl.ds(0, _CH)], semo).wait()

    bufs = ((rows0, o10, o20, o30, semg0, semo0),
            (rows1, o11, o21, o31, semg1, semo1))

    for p in range(_NB):
        start_gather(p, bufs[p][0], bufs[p][4])

    def ring_body(jj, carry):
        for p in range(_NB):
            rows, o1, o2, o3, semg, semo = bufs[p]
            j = _NB * jj + p
            wait_gather(rows, semg)

            @pl.when(jj > 0)
            def _():
                wait_out(o1, o2, o3, semo)

            compute(j, rows, o1, o2, o3)
            store_out(j, o1, o2, o3, semo)

            @pl.when(j + _NB < _NCH)
            def _():
                start_gather(j + _NB, rows, semg)

        return carry

    lax.fori_loop(0, _NCH // _NB, ring_body, 0)
    for j in range((_NCH // _NB) * _NB, _NCH):
        p = j % _NB
        rows, o1, o2, o3, semg, semo = bufs[p]
        wait_gather(rows, semg)
        wait_out(o1, o2, o3, semo)
        compute(j, rows, o1, o2, o3)
        store_out(j, o1, o2, o3, semo)
    for p in range(_NB):
        rows, o1, o2, o3, semg, semo = bufs[p]
        wait_out(o1, o2, o3, semo)


_agg = functools.partial(
    pl.kernel,
    mesh=plsc.VectorSubcoreMesh(core_axis_name="c", subcore_axis_name="s"),
    out_type=[jax.ShapeDtypeStruct((_NPAD, _D), jnp.float32)] * 3,
    scratch_types=[
        pltpu.VMEM_SHARED((_ASH, _D), jnp.float32),
        pltpu.VMEM((_NCH, _EPC), jnp.int32),
        pltpu.VMEM((_NCH, _EPC), jnp.float32),
        pltpu.VMEM((_EPC, _D), jnp.float32),
        pltpu.VMEM((_EPC, _D), jnp.float32),
        pltpu.VMEM((_CH, _D), jnp.float32),
        pltpu.VMEM((_CH, _D), jnp.float32),
        pltpu.VMEM((_CH, _D), jnp.float32),
        pltpu.VMEM((_CH, _D), jnp.float32),
        pltpu.VMEM((_CH, _D), jnp.float32),
        pltpu.VMEM((_CH, _D), jnp.float32),
        pltpu.SemaphoreType.DMA,
        pltpu.SemaphoreType.DMA,
        pltpu.SemaphoreType.DMA,
        pltpu.SemaphoreType.DMA,
    ],
)(_agg_body)


def _post_core(h_ref, bb_ref, s1_ref, s2_ref, s3_ref, sw_ref, snorm_ref,
               pw_ref, pb_ref, g_ref, beta_ref):
    h = h_ref[...]
    bb = bb_ref[...]
    s1 = s1_ref[0:_N, :]
    s2 = s2_ref[0:_N, :]
    s3 = s3_ref[0:_N, :]
    sw = sw_ref[...]
    pw = pw_ref[...]
    sum_e = s1 + 16.0 * bb
    max_agg = s2 + bb
    dir_agg = jnp.abs(s3 + sw * bb - sw * h)
    p_ms = pw[128:256, :] * (1.0 / 16.0) + pw[256:384, :]
    x = (jnp.dot(h, pw[0:128, :], preferred_element_type=jnp.float32)
         + jnp.dot(sum_e, p_ms, preferred_element_type=jnp.float32)
         + jnp.dot(max_agg, pw[384:512, :], preferred_element_type=jnp.float32)
         + jnp.dot(dir_agg, pw[512:640, :], preferred_element_type=jnp.float32)
         + pb_ref[...])
    x = x * snorm_ref[...]
    mu = jnp.mean(x, axis=0, keepdims=True)
    xc = x - mu
    var = jnp.mean(xc * xc, axis=0, keepdims=True)
    y = xc * lax.rsqrt(var + 1e-5) * g_ref[...] + beta_ref[...]
    return jnp.maximum(y, 0.0)


def _post_body(h_ref, bb_ref, s1_ref, s2_ref, s3_ref, sw_ref, snorm_ref,
               pw_ref, pb_ref, g_ref, beta_ref, out_ref):
    out_ref[...] = _post_core(h_ref, bb_ref, s1_ref, s2_ref, s3_ref, sw_ref,
                              snorm_ref, pw_ref, pb_ref, g_ref, beta_ref)


_post = pl.pallas_call(
    _post_body,
    out_shape=jax.ShapeDtypeStruct((_N, _D), jnp.float32),
)


def _pre_body(h_ref, w1_ref, w2_ref, b_ref, a_ref, bb_ref):
    h = h_ref[...]
    a_ref[0:_N, :] = jnp.dot(h, w1_ref[...], preferred_element_type=jnp.float32)
    bb_ref[...] = jnp.dot(h, w2_ref[...], preferred_element_type=jnp.float32) + b_ref[...]


_pre = pl.pallas_call(
    _pre_body,
    out_shape=(
        jax.ShapeDtypeStruct((_ASH, _D), jnp.float32),
        jax.ShapeDtypeStruct((_N, _D), jnp.float32),
    ),
)


def _post_ro_body(h_ref, bb_ref, s1_ref, s2_ref, s3_ref, sw_ref, snorm_ref,
                  pw_ref, pb_ref, g_ref, beta_ref,
                  w0_ref, b0_ref, w1_ref, b1_ref, w2_ref, b2_ref, out_ref):
    hn = _post_core(h_ref, bb_ref, s1_ref, s2_ref, s3_ref, sw_ref, snorm_ref,
                    pw_ref, pb_ref, g_ref, beta_ref)
    hg = jnp.mean(hn, axis=0, keepdims=True)
    y = jnp.maximum(jnp.dot(hg, w0_ref[...], preferred_element_type=jnp.float32) + b0_ref[...], 0.0)
    y = jnp.maximum(jnp.dot(y, w1_ref[...], preferred_element_type=jnp.float32) + b1_ref[...], 0.0)
    out_ref[...] = jnp.dot(y, w2_ref[...], preferred_element_type=jnp.float32) + b2_ref[...]


_post_ro = pl.pallas_call(
    _post_ro_body,
    out_shape=jax.ShapeDtypeStruct((1, 128), jnp.float32),
)


def kernel(h, edge_index, eig, snorm_n, atom_table, pre_W, pre_b, post_W,
           post_b, bn_g, bn_b, ro_W0, ro_b0, ro_W1, ro_b1, ro_W2, ro_b2):
    hidx = h.astype(jnp.int32)
    src = edge_index[0].astype(jnp.int32)
    srcp = jnp.pad(src, (0, _EPAD - src.shape[0])).reshape(_EROWS, _EPC)
    w_col = eig[:, 1].reshape(_N, _DEG)
    table_pad = jnp.pad(atom_table, ((0, 256 - _TOTAL), (0, 0)))

    def wslice(l):
        return (pre_W[l, :_D, :], pre_W[l, _D:, :], pre_b[l].reshape(1, _D))

    w1, w2, b = wslice(0)
    hcur, ew, sw, a, bb = _embed_pre(hidx, table_pad, w_col, w1, w2, b)
    ewp = jnp.pad(ew.reshape(-1), (0, _EPAD - _N * _DEG)).reshape(_EROWS, _EPC)

    for l in range(_L - 1):
        s1, s2, s3 = _agg(a, srcp, ewp)
        hcur = _post(hcur, bb, s1, s2, s3, sw, snorm_n, post_W[l],
                     post_b[l].reshape(1, _D), bn_g[l].reshape(1, _D),
                     bn_b[l].reshape(1, _D))
        w1, w2, b = wslice(l + 1)
        a, bb = _pre(hcur, w1, w2, b)

    s1, s2, s3 = _agg(a, srcp, ewp)
    l = _L - 1
    return _post_ro(hcur, bb, s1, s2, s3, sw, snorm_n, post_W[l],
                    post_b[l].reshape(1, _D), bn_g[l].reshape(1, _D),
                    bn_b[l].reshape(1, _D),
                    ro_W0, ro_b0.reshape(1, -1), ro_W1,
                    ro_b1.reshape(1, -1), ro_W2, ro_b2.reshape(1, -1))
